# baseline (speedup 1.0000x reference)
import jax
import jax.numpy as jnp
from jax.experimental import pallas as pl

_B, _N, _CIN, _D, _Q, _NCLS = 4, 30000, 4, 32, 100, 20
_N2 = _N // 16
_CF = 3 + _CIN
_BLK = 3000
_NB = _N // _BLK

_INV_SQRT_D = 1.0 / float(_D) ** 0.5


def _coarse(cfr_ref, Win_ref, bin_ref, W2_ref, b2_ref, Wsem_ref, q_ref,
            Wcls_ref, logits_ref, sem_ref, qf_ref):
    c2f2 = cfr_ref[0][:, :_CF]
    x2 = jnp.maximum(c2f2 @ Win_ref[...] + bin_ref[...], 0.0)
    l2 = jnp.maximum(x2 @ W2_ref[...] + b2_ref[...], 0.0)
    sem_ref[0] = l2 @ Wsem_ref[...]
    scores = jax.lax.dot_general(
        q_ref[...], l2, (((1,), (1,)), ((), ()))) * _INV_SQRT_D
    scores = scores - jnp.max(scores, axis=-1, keepdims=True)
    e = jnp.exp(scores)
    attn = e / jnp.sum(e, axis=-1, keepdims=True)
    qf = attn @ l2
    qf_ref[0] = qf
    logits_ref[0] = qf @ Wcls_ref[...]


def _masks(cf_ref, Win_ref, bin_ref, W0_ref, b0_ref, qf_ref, masks_ref):
    x = jnp.maximum(cf_ref[0] @ Win_ref[...] + bin_ref[...], 0.0)
    l0 = jnp.maximum(x @ W0_ref[...] + b0_ref[...], 0.0)
    masks_ref[0] = jax.lax.dot_general(
        l0, qf_ref[0], (((1,), (1,)), ((), ())))


def kernel(coords, feats, W_in, b_in, W_lvl, b_lvl, W_sem, queries, W_cls):
    cf = jnp.concatenate([coords, feats], axis=-1)
    cfr = cf.reshape(_B, _N2, 16 * _CF)
    b_in2 = b_in.reshape(1, _D)
    W0, W2 = W_lvl[0], W_lvl[2]
    b0, b2 = b_lvl[0].reshape(1, _D), b_lvl[2].reshape(1, _D)

    full = lambda *shape: pl.BlockSpec(shape, lambda *_: (0,) * len(shape))
    per_b = lambda *shape: pl.BlockSpec(shape, lambda b, *_: (b, 0, 0))

    logits, sem, qf = pl.pallas_call(
        _coarse,
        grid=(_B,),
        in_specs=[
            per_b(1, _N2, 16 * _CF),
            full(_CF, _D),
            full(1, _D),
            full(_D, _D),
            full(1, _D),
            full(_D, _NCLS),
            full(_Q, _D),
            full(_D, _NCLS + 1),
        ],
        out_specs=[
            per_b(1, _Q, _NCLS + 1),
            per_b(1, _N2, _NCLS),
            per_b(1, _Q, _D),
        ],
        out_shape=[
            jax.ShapeDtypeStruct((_B, _Q, _NCLS + 1), jnp.float32),
            jax.ShapeDtypeStruct((_B, _N2, _NCLS), jnp.float32),
            jax.ShapeDtypeStruct((_B, _Q, _D), jnp.float32),
        ],
    )(cfr, W_in, b_in2, W2, b2, W_sem, queries, W_cls)

    masks = pl.pallas_call(
        _masks,
        grid=(_B, _NB),
        in_specs=[
            pl.BlockSpec((1, _BLK, _CF), lambda b, j: (b, j, 0)),
            full(_CF, _D),
            full(1, _D),
            full(_D, _D),
            full(1, _D),
            per_b(1, _Q, _D),
        ],
        out_specs=pl.BlockSpec((1, _BLK, _Q), lambda b, j: (b, j, 0)),
        out_shape=jax.ShapeDtypeStruct((_B, _N, _Q), jnp.float32),
    )(cf, W_in, b_in2, W0, b0, qf)

    return (logits, masks, sem)
